# Initial kernel scaffold; baseline (speedup 1.0000x reference)
import jax
import jax.numpy as jnp
from jax.experimental import pallas as pl


def kernel(x, edge_index, emb_table, W1, b1, W2, b2):
    raise NotImplementedError("write your pallas kernel here")


# speedup vs baseline: 15.3441x; 15.3441x over previous
import functools

import jax
import jax.numpy as jnp
from jax import lax
from jax.experimental import pallas as pl
from jax.experimental.pallas import tpu as pltpu
from jax.experimental.pallas import tpu_sc as plsc

N = 10000
D = 128
H2 = 256
E = 320000
NC, NS = 2, 16
NW = NC * NS
CK = 80
EPT = E // NW
CPT = EPT // CK
ZR = 624
VL = 16
_R = 1000

_mesh = plsc.VectorSubcoreMesh(core_axis_name="c", subcore_axis_name="s")


@functools.partial(
    pl.kernel,
    out_type=[
        jax.ShapeDtypeStruct((N, D), jnp.float32),
        jax.ShapeDtypeStruct((NW * N,), jnp.float32),
    ],
    mesh=_mesh,
    scratch_types=[
        pltpu.VMEM((EPT,), jnp.int32),
        pltpu.VMEM((N,), jnp.float32),
        pltpu.VMEM((CK,), jnp.int32),
        pltpu.VMEM((CK, D), jnp.float32),
        pltpu.SemaphoreType.DMA,
    ],
    compiler_params=pltpu.CompilerParams(needs_layout_passes=False),
)
def _sc_emb_deg(x1, dst1, emb, h_out, degp_out, dst_v, acc, x_v, rows_v, sem):
    c = lax.axis_index("c")
    s = lax.axis_index("s")
    wid = c * NS + s

    pltpu.sync_copy(dst1.at[pl.ds(wid * EPT, EPT)], dst_v)

    def zero_body(j, carry):
        acc[pl.ds(VL * j, VL)] = jnp.zeros((VL,), jnp.float32)
        return carry

    lax.fori_loop(0, N // VL, zero_body, 0)

    ones = jnp.ones((VL,), jnp.float32)

    def deg_body(j, carry):
        iv = dst_v[pl.ds(VL * j, VL)]
        plsc.addupdate_scatter(acc, [iv], ones)
        return carry

    lax.fori_loop(0, EPT // VL, deg_body, 0)

    def wr_body(b, carry):
        pltpu.sync_copy(acc.at[pl.ds(b * _R, _R)],
                        degp_out.at[pl.ds((b * NW + wid) * _R, _R)])
        return carry

    lax.fori_loop(0, N // _R, wr_body, 0)

    @pl.when(wid < 25)
    def _():
        def emb_body(j, carry):
            pltpu.sync_copy(x1.at[pl.ds(wid * 400 + j * CK, CK)], x_v)
            pltpu.async_copy(emb.at[x_v], rows_v, sem).wait()
            pltpu.sync_copy(rows_v, h_out.at[pl.ds(wid * 400 + j * CK, CK)])
            return carry

        lax.fori_loop(0, 5, emb_body, 0)


@functools.partial(
    pl.kernel,
    out_type=jax.ShapeDtypeStruct((NC, N, D), jnp.float32),
    mesh=_mesh,
    scratch_types=[
        pltpu.VMEM_SHARED((N, D), jnp.float32),
        pltpu.VMEM((CK,), jnp.int32),
        pltpu.VMEM((CK,), jnp.int32),
        pltpu.VMEM((CK, D), jnp.float32),
        pltpu.SemaphoreType.DMA,
    ],
)
def _sc_segsum(hs, src1, dst1, zf, out, acc, src_v, dst_v, rows_v, sem):
    c = lax.axis_index("c")
    s = lax.axis_index("s")
    wid = c * NS + s

    pltpu.sync_copy(zf.at[pl.ds(s * ZR, ZR)], acc.at[pl.ds(s * ZR, ZR)])

    @pl.when(s == NS - 1)
    def _():
        pltpu.sync_copy(zf.at[pl.ds(NS * ZR, N - NS * ZR)],
                        acc.at[pl.ds(NS * ZR, N - NS * ZR)])

    plsc.subcore_barrier()

    def body(j, carry):
        pltpu.sync_copy(src1.at[pl.ds(wid * EPT + j * CK, CK)], src_v)
        pltpu.sync_copy(dst1.at[pl.ds(wid * EPT + j * CK, CK)], dst_v)
        pltpu.async_copy(hs.at[src_v], rows_v, sem).wait()
        pltpu.sync_copy(rows_v, acc.at[dst_v], add=True)
        return carry

    lax.fori_loop(0, CPT, body, 0)

    plsc.subcore_barrier()
    pltpu.sync_copy(acc.at[pl.ds(s * ZR, ZR)], out.at[c, pl.ds(s * ZR, ZR)])

    @pl.when(s == NS - 1)
    def _():
        pltpu.sync_copy(acc.at[pl.ds(NS * ZR, N - NS * ZR)],
                        out.at[c, pl.ds(NS * ZR, N - NS * ZR)])


def _tc_scale_body(dps, h, hs1, dinv_out):
    dp = dps[...].reshape(NW, _R)
    ones = jnp.ones((NW, 1), jnp.float32)
    deg = lax.dot_general(dp, ones, (((0,), (0,)), ((), ())),
                          preferred_element_type=jnp.float32) + 1.0
    dcol = lax.rsqrt(deg)
    hs1[...] = h[...] * dcol
    dinv_out[...] = dcol


def _tc_mid_body(dinv, t0, t1, hs1, w1, b1, w2, hs2):
    dcol = dinv[...]
    aggh = (t0[...] + t1[...] + hs1[...]) * dcol
    o1 = jnp.dot(aggh, w1[...], preferred_element_type=jnp.float32) + b1[...]
    h2 = jnp.maximum(o1, 0.0)
    xw2 = jnp.dot(h2, w2[...], preferred_element_type=jnp.float32)
    hs2[...] = xw2 * dcol


def _tc_out_body(dinv, t0, t1, hs2, b2, out):
    out[...] = (t0[...] + t1[...] + hs2[...]) * dinv[...] + b2[...]


def _row_spec(w):
    return pl.BlockSpec((_R, w), lambda i: (i, 0))


def _full_spec(r, w):
    return pl.BlockSpec((r, w), lambda i: (0, 0))


def _tc_scale(dps, h):
    return pl.pallas_call(
        _tc_scale_body,
        grid=(N // _R,),
        in_specs=[pl.BlockSpec((1, NW, _R), lambda i: (i, 0, 0)),
                  _row_spec(D)],
        out_specs=[_row_spec(D), _row_spec(1)],
        out_shape=[jax.ShapeDtypeStruct((N, D), jnp.float32),
                   jax.ShapeDtypeStruct((N, 1), jnp.float32)],
    )(dps, h)


def _tc_mid(dinv, t0, t1, hs1, w1, b1, w2):
    return pl.pallas_call(
        _tc_mid_body,
        grid=(N // _R,),
        in_specs=[_row_spec(1), _row_spec(D), _row_spec(D), _row_spec(D),
                  _full_spec(D, H2), _full_spec(1, H2), _full_spec(H2, D)],
        out_specs=_row_spec(D),
        out_shape=jax.ShapeDtypeStruct((N, D), jnp.float32),
    )(dinv, t0, t1, hs1, w1, b1, w2)


def _tc_out(dinv, t0, t1, hs2, b2):
    return pl.pallas_call(
        _tc_out_body,
        grid=(N // _R,),
        in_specs=[_row_spec(1), _row_spec(D), _row_spec(D), _row_spec(D),
                  _full_spec(1, D)],
        out_specs=_row_spec(D),
        out_shape=jax.ShapeDtypeStruct((N, D), jnp.float32),
    )(dinv, t0, t1, hs2, b2)


def kernel(x, edge_index, emb_table, W1, b1, W2, b2):
    src1 = edge_index[0]
    dst1 = edge_index[1]
    x1 = x[:, 0]
    zf = jnp.zeros((N, D), jnp.float32)

    h, degp = _sc_emb_deg(x1, dst1, emb_table)
    hs1, dinv = _tc_scale(degp.reshape(N // _R, NW, _R), h)

    t1 = _sc_segsum(hs1, src1, dst1, zf)
    hs2 = _tc_mid(dinv, t1[0], t1[1], hs1, W1, b1.reshape(1, H2), W2)
    t2 = _sc_segsum(hs2, src1, dst1, zf)
    return _tc_out(dinv, t2[0], t2[1], hs2, b2.reshape(1, D))


# speedup vs baseline: 31.9943x; 2.0851x over previous
import functools

import jax
import jax.numpy as jnp
from jax import lax
from jax.experimental import pallas as pl
from jax.experimental.pallas import tpu as pltpu
from jax.experimental.pallas import tpu_sc as plsc

N = 10000
D = 128
H2 = 256
E = 320000
NC, NS = 2, 16
NW = NC * NS
CK = 80
EPT = E // NW
CPT = EPT // CK
ZR = 624
VL = 16
_R = 1000

_mesh = plsc.VectorSubcoreMesh(core_axis_name="c", subcore_axis_name="s")


@functools.partial(
    pl.kernel,
    out_type=[
        jax.ShapeDtypeStruct((N, D), jnp.float32),
        jax.ShapeDtypeStruct((NW * N,), jnp.float32),
    ],
    mesh=_mesh,
    scratch_types=[
        pltpu.VMEM((EPT,), jnp.int32),
        pltpu.VMEM((N,), jnp.float32),
        pltpu.VMEM((CK,), jnp.int32),
        pltpu.VMEM((CK, D), jnp.float32),
        pltpu.SemaphoreType.DMA,
    ],
    compiler_params=pltpu.CompilerParams(needs_layout_passes=False),
)
def _sc_emb_deg(x1, dst1, emb, h_out, degp_out, dst_v, acc, x_v, rows_v, sem):
    c = lax.axis_index("c")
    s = lax.axis_index("s")
    wid = c * NS + s

    pltpu.sync_copy(dst1.at[pl.ds(wid * EPT, EPT)], dst_v)

    def zero_body(j, carry):
        acc[pl.ds(VL * j, VL)] = jnp.zeros((VL,), jnp.float32)
        return carry

    lax.fori_loop(0, N // VL, zero_body, 0)

    ones = jnp.ones((VL,), jnp.float32)

    def deg_body(j, carry):
        iv = dst_v[pl.ds(VL * j, VL)]
        plsc.addupdate_scatter(acc, [iv], ones)
        return carry

    lax.fori_loop(0, EPT // VL, deg_body, 0)

    def wr_body(b, carry):
        pltpu.sync_copy(acc.at[pl.ds(b * _R, _R)],
                        degp_out.at[pl.ds((b * NW + wid) * _R, _R)])
        return carry

    lax.fori_loop(0, N // _R, wr_body, 0)

    @pl.when(wid < 25)
    def _():
        def emb_body(j, carry):
            pltpu.sync_copy(x1.at[pl.ds(wid * 400 + j * CK, CK)], x_v)
            pltpu.async_copy(emb.at[x_v], rows_v, sem).wait()
            pltpu.sync_copy(rows_v, h_out.at[pl.ds(wid * 400 + j * CK, CK)])
            return carry

        lax.fori_loop(0, 5, emb_body, 0)


CK2 = 80
CPT2 = EPT // CK2
NB = 2


@functools.partial(
    pl.kernel,
    out_type=jax.ShapeDtypeStruct((NC, N, D), jnp.float32),
    mesh=_mesh,
    scratch_types=[
        pltpu.VMEM_SHARED((N, D), jnp.float32),
        pltpu.VMEM((EPT,), jnp.int32),
        pltpu.VMEM((CPT2, CK2), jnp.int32),
        pltpu.VMEM((CK2, D), jnp.float32),
        pltpu.VMEM((CK2, D), jnp.float32),
        pltpu.SemaphoreType.DMA,
        pltpu.SemaphoreType.DMA,
    ],
)
def _sc_segsum(hs, src1, dst3, zf, out, acc, srcs, dsts, rb0, rb1,
               sem0, sem1):
    c = lax.axis_index("c")
    s = lax.axis_index("s")
    wid = c * NS + s
    bufs = (rb0, rb1)
    sems = (sem0, sem1)

    pltpu.sync_copy(src1.at[pl.ds(wid * EPT, EPT)], srcs)
    pltpu.sync_copy(dst3.at[wid], dsts)

    pltpu.sync_copy(zf.at[pl.ds(s * ZR, ZR)], acc.at[pl.ds(s * ZR, ZR)])

    @pl.when(s == NS - 1)
    def _():
        pltpu.sync_copy(zf.at[pl.ds(NS * ZR, N - NS * ZR)],
                        acc.at[pl.ds(NS * ZR, N - NS * ZR)])

    plsc.subcore_barrier()

    for b in range(NB):
        pltpu.async_copy(hs.at[srcs.at[pl.ds(b * CK2, CK2)]], bufs[b],
                         sems[b])

    def body(g, carry):
        for b in range(NB):
            j = g * NB + b
            pltpu.make_async_copy(hs.at[srcs.at[pl.ds(j * CK2, CK2)]],
                                  bufs[b], sems[b]).wait()
            pltpu.sync_copy(bufs[b], acc.at[dsts.at[j]], add=True)

            @pl.when(j + NB < CPT2)
            def _():
                pltpu.async_copy(hs.at[srcs.at[pl.ds((j + NB) * CK2, CK2)]],
                                 bufs[b], sems[b])
        return carry

    lax.fori_loop(0, CPT2 // NB, body, 0)

    for j in range(NB * (CPT2 // NB), CPT2):
        b = j % NB
        pltpu.make_async_copy(hs.at[srcs.at[pl.ds(j * CK2, CK2)]],
                              bufs[b], sems[b]).wait()
        pltpu.sync_copy(bufs[b], acc.at[dsts.at[j]], add=True)

    plsc.subcore_barrier()
    pltpu.sync_copy(acc.at[pl.ds(s * ZR, ZR)], out.at[c, pl.ds(s * ZR, ZR)])

    @pl.when(s == NS - 1)
    def _():
        pltpu.sync_copy(acc.at[pl.ds(NS * ZR, N - NS * ZR)],
                        out.at[c, pl.ds(NS * ZR, N - NS * ZR)])


def _tc_scale_body(dps, h, hs1, dinv_out):
    dp = dps[...].reshape(NW, _R)
    ones = jnp.ones((NW, 1), jnp.float32)
    deg = lax.dot_general(dp, ones, (((0,), (0,)), ((), ())),
                          preferred_element_type=jnp.float32) + 1.0
    dcol = lax.rsqrt(deg)
    hs1[...] = h[...] * dcol
    dinv_out[...] = dcol


def _tc_mid_body(dinv, t0, t1, hs1, w1, b1, w2, hs2):
    dcol = dinv[...]
    aggh = (t0[...] + t1[...] + hs1[...]) * dcol
    o1 = jnp.dot(aggh, w1[...], preferred_element_type=jnp.float32) + b1[...]
    h2 = jnp.maximum(o1, 0.0)
    xw2 = jnp.dot(h2, w2[...], preferred_element_type=jnp.float32)
    hs2[...] = xw2 * dcol


def _tc_out_body(dinv, t0, t1, hs2, b2, out):
    out[...] = (t0[...] + t1[...] + hs2[...]) * dinv[...] + b2[...]


def _row_spec(w):
    return pl.BlockSpec((_R, w), lambda i: (i, 0))


def _full_spec(r, w):
    return pl.BlockSpec((r, w), lambda i: (0, 0))


def _tc_scale(dps, h):
    return pl.pallas_call(
        _tc_scale_body,
        grid=(N // _R,),
        in_specs=[pl.BlockSpec((1, NW, _R), lambda i: (i, 0, 0)),
                  _row_spec(D)],
        out_specs=[_row_spec(D), _row_spec(1)],
        out_shape=[jax.ShapeDtypeStruct((N, D), jnp.float32),
                   jax.ShapeDtypeStruct((N, 1), jnp.float32)],
    )(dps, h)


def _tc_mid(dinv, t0, t1, hs1, w1, b1, w2):
    return pl.pallas_call(
        _tc_mid_body,
        grid=(N // _R,),
        in_specs=[_row_spec(1), _row_spec(D), _row_spec(D), _row_spec(D),
                  _full_spec(D, H2), _full_spec(1, H2), _full_spec(H2, D)],
        out_specs=_row_spec(D),
        out_shape=jax.ShapeDtypeStruct((N, D), jnp.float32),
    )(dinv, t0, t1, hs1, w1, b1, w2)


def _tc_out(dinv, t0, t1, hs2, b2):
    return pl.pallas_call(
        _tc_out_body,
        grid=(N // _R,),
        in_specs=[_row_spec(1), _row_spec(D), _row_spec(D), _row_spec(D),
                  _full_spec(1, D)],
        out_specs=_row_spec(D),
        out_shape=jax.ShapeDtypeStruct((N, D), jnp.float32),
    )(dinv, t0, t1, hs2, b2)


def kernel(x, edge_index, emb_table, W1, b1, W2, b2):
    src1 = edge_index[0]
    dst1 = edge_index[1]
    dst3 = edge_index[1].reshape(NW, CPT2, CK2)
    x1 = x[:, 0]
    zf = jnp.zeros((N, D), jnp.float32)

    h, degp = _sc_emb_deg(x1, dst1, emb_table)
    hs1, dinv = _tc_scale(degp.reshape(N // _R, NW, _R), h)

    t1 = _sc_segsum(hs1, src1, dst3, zf)
    hs2 = _tc_mid(dinv, t1[0], t1[1], hs1, W1, b1.reshape(1, H2), W2)
    t2 = _sc_segsum(hs2, src1, dst3, zf)
    return _tc_out(dinv, t2[0], t2[1], hs2, b2.reshape(1, D))


# speedup vs baseline: 35.9567x; 1.1238x over previous
import functools

import jax
import jax.numpy as jnp
from jax import lax
from jax.experimental import pallas as pl
from jax.experimental.pallas import tpu as pltpu
from jax.experimental.pallas import tpu_sc as plsc

N = 10000
D = 128
H2 = 256
E = 320000
NC, NS = 2, 16
NW = NC * NS
CK = 80
EPT = E // NW
CPT = EPT // CK
ZR = 624
VL = 16
_R = 1000

_mesh = plsc.VectorSubcoreMesh(core_axis_name="c", subcore_axis_name="s")


@functools.partial(
    pl.kernel,
    out_type=[
        jax.ShapeDtypeStruct((N, D), jnp.float32),
        jax.ShapeDtypeStruct((NW * N,), jnp.float32),
    ],
    mesh=_mesh,
    scratch_types=[
        pltpu.VMEM((EPT,), jnp.int32),
        pltpu.VMEM((N,), jnp.float32),
        pltpu.VMEM((CK,), jnp.int32),
        pltpu.VMEM((CK, D), jnp.float32),
        pltpu.SemaphoreType.DMA,
    ],
    compiler_params=pltpu.CompilerParams(needs_layout_passes=False),
)
def _sc_emb_deg(x1, dst1, emb, h_out, degp_out, dst_v, acc, x_v, rows_v, sem):
    c = lax.axis_index("c")
    s = lax.axis_index("s")
    wid = c * NS + s

    pltpu.sync_copy(dst1.at[pl.ds(wid * EPT, EPT)], dst_v)

    def zero_body(j, carry):
        acc[pl.ds(VL * j, VL)] = jnp.zeros((VL,), jnp.float32)
        return carry

    lax.fori_loop(0, N // VL, zero_body, 0)

    ones = jnp.ones((VL,), jnp.float32)

    def deg_body(j, carry):
        iv = dst_v[pl.ds(VL * j, VL)]
        plsc.addupdate_scatter(acc, [iv], ones)
        return carry

    lax.fori_loop(0, EPT // VL, deg_body, 0)

    def wr_body(b, carry):
        pltpu.sync_copy(acc.at[pl.ds(b * _R, _R)],
                        degp_out.at[pl.ds((b * NW + wid) * _R, _R)])
        return carry

    lax.fori_loop(0, N // _R, wr_body, 0)

    @pl.when(wid < 25)
    def _():
        def emb_body(j, carry):
            pltpu.sync_copy(x1.at[pl.ds(wid * 400 + j * CK, CK)], x_v)
            pltpu.async_copy(emb.at[x_v], rows_v, sem).wait()
            pltpu.sync_copy(rows_v, h_out.at[pl.ds(wid * 400 + j * CK, CK)])
            return carry

        lax.fori_loop(0, 5, emb_body, 0)


CK2 = 80
CPT2 = EPT // CK2
PH = 64
NB = 3


@functools.partial(
    pl.kernel,
    out_type=jax.ShapeDtypeStruct((NC, N, D), jnp.float32),
    mesh=_mesh,
    scratch_types=[
        pltpu.VMEM_SHARED((N, D), jnp.float32),
        pltpu.VMEM((PH * CK2,), jnp.int32),
        pltpu.VMEM((PH, CK2), jnp.int32),
        pltpu.VMEM((CK2, D), jnp.float32),
        pltpu.VMEM((CK2, D), jnp.float32),
        pltpu.VMEM((CK2, D), jnp.float32),
        pltpu.SemaphoreType.DMA,
        pltpu.SemaphoreType.DMA,
        pltpu.SemaphoreType.DMA,
    ],
)
def _sc_segsum(hs, src1, dst3p, zf, out, acc, srcs, dsts, rb0, rb1, rb2,
               sem0, sem1, sem2):
    c = lax.axis_index("c")
    s = lax.axis_index("s")
    wid = c * NS + s
    bufs = (rb0, rb1, rb2)
    sems = (sem0, sem1, sem2)

    def fire(j, b):
        pltpu.async_copy(hs.at[srcs.at[pl.ds(j * CK2, CK2)]], bufs[b],
                         sems[b])

    def wait_scat(j, b):
        pltpu.make_async_copy(hs.at[srcs.at[pl.ds(j * CK2, CK2)]],
                              bufs[b], sems[b]).wait()
        pltpu.sync_copy(bufs[b], acc.at[dsts.at[j]], add=True)

    def run_phase(nchunks):
        for b in range(NB):
            fire(b, b)

        def body(g, carry):
            for b in range(NB):
                j = g * NB + b
                wait_scat(j, b)
                fire(j + NB, b)
            return carry

        lax.fori_loop(0, (nchunks - 4) // NB, body, 0)
        j0 = nchunks - 4
        wait_scat(j0, j0 % NB)
        fire(nchunks - 1, (nchunks - 1) % NB)
        for j in range(j0 + 1, nchunks):
            wait_scat(j, j % NB)

    pltpu.sync_copy(src1.at[pl.ds(wid * EPT, PH * CK2)], srcs)
    pltpu.sync_copy(dst3p.at[wid, pl.ds(0, PH)], dsts)

    pltpu.sync_copy(zf.at[pl.ds(s * ZR, ZR)], acc.at[pl.ds(s * ZR, ZR)])

    @pl.when(s == NS - 1)
    def _():
        pltpu.sync_copy(zf.at[pl.ds(NS * ZR, N - NS * ZR)],
                        acc.at[pl.ds(NS * ZR, N - NS * ZR)])

    plsc.subcore_barrier()

    run_phase(PH)

    pltpu.sync_copy(src1.at[pl.ds(wid * EPT + PH * CK2, EPT - PH * CK2)],
                    srcs.at[pl.ds(0, EPT - PH * CK2)])
    pltpu.sync_copy(dst3p.at[wid, pl.ds(PH, PH)], dsts)

    run_phase(CPT2 - PH)

    plsc.subcore_barrier()
    pltpu.sync_copy(acc.at[pl.ds(s * ZR, ZR)], out.at[c, pl.ds(s * ZR, ZR)])

    @pl.when(s == NS - 1)
    def _():
        pltpu.sync_copy(acc.at[pl.ds(NS * ZR, N - NS * ZR)],
                        out.at[c, pl.ds(NS * ZR, N - NS * ZR)])


def _tc_scale_body(dps, h, hs1, dinv_out):
    dp = dps[...].reshape(NW, _R)
    ones = jnp.ones((NW, 1), jnp.float32)
    deg = lax.dot_general(dp, ones, (((0,), (0,)), ((), ())),
                          preferred_element_type=jnp.float32) + 1.0
    dcol = lax.rsqrt(deg)
    hs1[...] = h[...] * dcol
    dinv_out[...] = dcol


def _tc_mid_body(dinv, t0, t1, hs1, w1, b1, w2, hs2):
    dcol = dinv[...]
    aggh = (t0[...] + t1[...] + hs1[...]) * dcol
    o1 = jnp.dot(aggh, w1[...], preferred_element_type=jnp.float32) + b1[...]
    h2 = jnp.maximum(o1, 0.0)
    xw2 = jnp.dot(h2, w2[...], preferred_element_type=jnp.float32)
    hs2[...] = xw2 * dcol


def _tc_out_body(dinv, t0, t1, hs2, b2, out):
    out[...] = (t0[...] + t1[...] + hs2[...]) * dinv[...] + b2[...]


def _row_spec(w):
    return pl.BlockSpec((_R, w), lambda i: (i, 0))


def _full_spec(r, w):
    return pl.BlockSpec((r, w), lambda i: (0, 0))


def _tc_scale(dps, h):
    return pl.pallas_call(
        _tc_scale_body,
        grid=(N // _R,),
        in_specs=[pl.BlockSpec((1, NW, _R), lambda i: (i, 0, 0)),
                  _row_spec(D)],
        out_specs=[_row_spec(D), _row_spec(1)],
        out_shape=[jax.ShapeDtypeStruct((N, D), jnp.float32),
                   jax.ShapeDtypeStruct((N, 1), jnp.float32)],
    )(dps, h)


def _tc_mid(dinv, t0, t1, hs1, w1, b1, w2):
    return pl.pallas_call(
        _tc_mid_body,
        grid=(N // _R,),
        in_specs=[_row_spec(1), _row_spec(D), _row_spec(D), _row_spec(D),
                  _full_spec(D, H2), _full_spec(1, H2), _full_spec(H2, D)],
        out_specs=_row_spec(D),
        out_shape=jax.ShapeDtypeStruct((N, D), jnp.float32),
    )(dinv, t0, t1, hs1, w1, b1, w2)


def _tc_out(dinv, t0, t1, hs2, b2):
    return pl.pallas_call(
        _tc_out_body,
        grid=(N // _R,),
        in_specs=[_row_spec(1), _row_spec(D), _row_spec(D), _row_spec(D),
                  _full_spec(1, D)],
        out_specs=_row_spec(D),
        out_shape=jax.ShapeDtypeStruct((N, D), jnp.float32),
    )(dinv, t0, t1, hs2, b2)


def kernel(x, edge_index, emb_table, W1, b1, W2, b2):
    src1 = edge_index[0]
    dst1 = edge_index[1]
    dst3p = jnp.pad(edge_index[1].reshape(NW, CPT2, CK2),
                    ((0, 0), (0, 2 * PH - CPT2), (0, 0)))
    x1 = x[:, 0]
    zf = jnp.zeros((N, D), jnp.float32)

    h, degp = _sc_emb_deg(x1, dst1, emb_table)
    hs1, dinv = _tc_scale(degp.reshape(N // _R, NW, _R), h)

    t1 = _sc_segsum(hs1, src1, dst3p, zf)
    hs2 = _tc_mid(dinv, t1[0], t1[1], hs1, W1, b1.reshape(1, H2), W2)
    t2 = _sc_segsum(hs2, src1, dst3p, zf)
    return _tc_out(dinv, t2[0], t2[1], hs2, b2.reshape(1, D))


# speedup vs baseline: 36.1847x; 1.0063x over previous
import functools

import jax
import jax.numpy as jnp
from jax import lax
from jax.experimental import pallas as pl
from jax.experimental.pallas import tpu as pltpu
from jax.experimental.pallas import tpu_sc as plsc

N = 10000
D = 128
H2 = 256
E = 320000
NC, NS = 2, 16
NW = NC * NS
CK = 80
EPT = E // NW
CPT = EPT // CK
ZR = 624
VL = 16
_R = 1000

_mesh = plsc.VectorSubcoreMesh(core_axis_name="c", subcore_axis_name="s")


@functools.partial(
    pl.kernel,
    out_type=[
        jax.ShapeDtypeStruct((N, D), jnp.float32),
        jax.ShapeDtypeStruct((NW * N,), jnp.float32),
    ],
    mesh=_mesh,
    scratch_types=[
        pltpu.VMEM((EPT,), jnp.int32),
        pltpu.VMEM((N,), jnp.float32),
        pltpu.VMEM((CK,), jnp.int32),
        pltpu.VMEM((CK, D), jnp.float32),
        pltpu.SemaphoreType.DMA,
    ],
    compiler_params=pltpu.CompilerParams(needs_layout_passes=False),
)
def _sc_emb_deg(x1, dst1, emb, h_out, degp_out, dst_v, acc, x_v, rows_v, sem):
    c = lax.axis_index("c")
    s = lax.axis_index("s")
    wid = c * NS + s

    pltpu.sync_copy(dst1.at[pl.ds(wid * EPT, EPT)], dst_v)

    def zero_body(j, carry):
        acc[pl.ds(VL * j, VL)] = jnp.zeros((VL,), jnp.float32)
        return carry

    lax.fori_loop(0, N // VL, zero_body, 0)

    ones = jnp.ones((VL,), jnp.float32)

    def deg_body(j, carry):
        iv = dst_v[pl.ds(VL * j, VL)]
        plsc.addupdate_scatter(acc, [iv], ones)
        return carry

    lax.fori_loop(0, EPT // VL, deg_body, 0)

    def wr_body(b, carry):
        pltpu.sync_copy(acc.at[pl.ds(b * _R, _R)],
                        degp_out.at[pl.ds((b * NW + wid) * _R, _R)])
        return carry

    lax.fori_loop(0, N // _R, wr_body, 0)

    @pl.when(wid < 25)
    def _():
        def emb_body(j, carry):
            pltpu.sync_copy(x1.at[pl.ds(wid * 400 + j * CK, CK)], x_v)
            pltpu.async_copy(emb.at[x_v], rows_v, sem).wait()
            pltpu.sync_copy(rows_v, h_out.at[pl.ds(wid * 400 + j * CK, CK)])
            return carry

        lax.fori_loop(0, 5, emb_body, 0)


CK2 = 80
CPT2 = EPT // CK2
PH = 64
NB = 3


@functools.partial(
    pl.kernel,
    out_type=jax.ShapeDtypeStruct((NC, N, D), jnp.float32),
    mesh=_mesh,
    scratch_types=[
        pltpu.VMEM_SHARED((N, D), jnp.float32),
        pltpu.VMEM((PH * CK2,), jnp.int32),
        pltpu.VMEM((PH, CK2), jnp.int32),
        pltpu.VMEM((CK2, D), jnp.float32),
        pltpu.VMEM((CK2, D), jnp.float32),
        pltpu.VMEM((CK2, D), jnp.float32),
        pltpu.SemaphoreType.DMA,
        pltpu.SemaphoreType.DMA,
        pltpu.SemaphoreType.DMA,
    ],
)
def _sc_segsum(hs, src1, dst3p, zf80, out, acc, srcs, dsts, rb0, rb1, rb2,
               sem0, sem1, sem2):
    c = lax.axis_index("c")
    s = lax.axis_index("s")
    wid = c * NS + s
    bufs = (rb0, rb1, rb2)
    sems = (sem0, sem1, sem2)

    def fire(j, b):
        pltpu.async_copy(hs.at[srcs.at[pl.ds(j * CK2, CK2)]], bufs[b],
                         sems[b])

    def wait_scat(j, b):
        pltpu.make_async_copy(hs.at[srcs.at[pl.ds(j * CK2, CK2)]],
                              bufs[b], sems[b]).wait()
        pltpu.sync_copy(bufs[b], acc.at[dsts.at[j]], add=True)

    def run_phase(nchunks):
        for b in range(NB):
            fire(b, b)

        def body(g, carry):
            for b in range(NB):
                j = g * NB + b
                wait_scat(j, b)
                fire(j + NB, b)
            return carry

        lax.fori_loop(0, (nchunks - 4) // NB, body, 0)
        j0 = nchunks - 4
        wait_scat(j0, j0 % NB)
        fire(nchunks - 1, (nchunks - 1) % NB)
        for j in range(j0 + 1, nchunks):
            wait_scat(j, j % NB)

    pltpu.sync_copy(src1.at[pl.ds(wid * EPT, PH * CK2)], srcs)
    pltpu.sync_copy(dst3p.at[wid, pl.ds(0, PH)], dsts)

    pltpu.sync_copy(zf80, rb0)
    for k in range(7):
        pltpu.sync_copy(rb0, acc.at[pl.ds(s * ZR + k * CK2, CK2)])
    pltpu.sync_copy(rb0.at[pl.ds(0, 64)], acc.at[pl.ds(s * ZR + 560, 64)])

    @pl.when(s == NS - 1)
    def _():
        pltpu.sync_copy(rb0.at[pl.ds(0, N - NS * ZR)],
                        acc.at[pl.ds(NS * ZR, N - NS * ZR)])

    plsc.subcore_barrier()

    run_phase(PH)

    pltpu.sync_copy(src1.at[pl.ds(wid * EPT + PH * CK2, EPT - PH * CK2)],
                    srcs.at[pl.ds(0, EPT - PH * CK2)])
    pltpu.sync_copy(dst3p.at[wid, pl.ds(PH, PH)], dsts)

    run_phase(CPT2 - PH)

    plsc.subcore_barrier()
    pltpu.sync_copy(acc.at[pl.ds(s * ZR, ZR)], out.at[c, pl.ds(s * ZR, ZR)])

    @pl.when(s == NS - 1)
    def _():
        pltpu.sync_copy(acc.at[pl.ds(NS * ZR, N - NS * ZR)],
                        out.at[c, pl.ds(NS * ZR, N - NS * ZR)])


def _tc_scale_body(dps, h, hs1, dinv_out):
    dp = dps[...].reshape(NW, _R)
    ones = jnp.ones((NW, 1), jnp.float32)
    deg = lax.dot_general(dp, ones, (((0,), (0,)), ((), ())),
                          preferred_element_type=jnp.float32) + 1.0
    dcol = lax.rsqrt(deg)
    hs1[...] = h[...] * dcol
    dinv_out[...] = dcol


def _tc_mid_body(dinv, t0, t1, hs1, w1, b1, w2, hs2):
    dcol = dinv[...]
    aggh = (t0[...] + t1[...] + hs1[...]) * dcol
    o1 = jnp.dot(aggh, w1[...], preferred_element_type=jnp.float32) + b1[...]
    h2 = jnp.maximum(o1, 0.0)
    xw2 = jnp.dot(h2, w2[...], preferred_element_type=jnp.float32)
    hs2[...] = xw2 * dcol


def _tc_out_body(dinv, t0, t1, hs2, b2, out):
    out[...] = (t0[...] + t1[...] + hs2[...]) * dinv[...] + b2[...]


def _row_spec(w):
    return pl.BlockSpec((_R, w), lambda i: (i, 0))


def _full_spec(r, w):
    return pl.BlockSpec((r, w), lambda i: (0, 0))


def _tc_scale(dps, h):
    return pl.pallas_call(
        _tc_scale_body,
        grid=(N // _R,),
        in_specs=[pl.BlockSpec((1, NW, _R), lambda i: (i, 0, 0)),
                  _row_spec(D)],
        out_specs=[_row_spec(D), _row_spec(1)],
        out_shape=[jax.ShapeDtypeStruct((N, D), jnp.float32),
                   jax.ShapeDtypeStruct((N, 1), jnp.float32)],
    )(dps, h)


def _tc_mid(dinv, t0, t1, hs1, w1, b1, w2):
    return pl.pallas_call(
        _tc_mid_body,
        grid=(N // _R,),
        in_specs=[_row_spec(1), _row_spec(D), _row_spec(D), _row_spec(D),
                  _full_spec(D, H2), _full_spec(1, H2), _full_spec(H2, D)],
        out_specs=_row_spec(D),
        out_shape=jax.ShapeDtypeStruct((N, D), jnp.float32),
    )(dinv, t0, t1, hs1, w1, b1, w2)


def _tc_out(dinv, t0, t1, hs2, b2):
    return pl.pallas_call(
        _tc_out_body,
        grid=(N // _R,),
        in_specs=[_row_spec(1), _row_spec(D), _row_spec(D), _row_spec(D),
                  _full_spec(1, D)],
        out_specs=_row_spec(D),
        out_shape=jax.ShapeDtypeStruct((N, D), jnp.float32),
    )(dinv, t0, t1, hs2, b2)


def kernel(x, edge_index, emb_table, W1, b1, W2, b2):
    src1 = edge_index[0]
    dst1 = edge_index[1]
    dst3p = jnp.pad(edge_index[1].reshape(NW, CPT2, CK2),
                    ((0, 0), (0, 2 * PH - CPT2), (0, 0)))
    x1 = x[:, 0]
    zf80 = jnp.zeros((CK2, D), jnp.float32)

    h, degp = _sc_emb_deg(x1, dst1, emb_table)
    hs1, dinv = _tc_scale(degp.reshape(N // _R, NW, _R), h)

    t1 = _sc_segsum(hs1, src1, dst3p, zf80)
    hs2 = _tc_mid(dinv, t1[0], t1[1], hs1, W1, b1.reshape(1, H2), W2)
    t2 = _sc_segsum(hs2, src1, dst3p, zf80)
    return _tc_out(dinv, t2[0], t2[1], hs2, b2.reshape(1, D))


# speedup vs baseline: 36.4475x; 1.0073x over previous
import functools

import jax
import jax.numpy as jnp
from jax import lax
from jax.experimental import pallas as pl
from jax.experimental.pallas import tpu as pltpu
from jax.experimental.pallas import tpu_sc as plsc

N = 10000
D = 128
H2 = 256
E = 320000
NC, NS = 2, 16
NW = NC * NS
CK = 80
EPT = E // NW
CPT = EPT // CK
ZR = 624
VL = 16
_R = 1000

_mesh = plsc.VectorSubcoreMesh(core_axis_name="c", subcore_axis_name="s")


@functools.partial(
    pl.kernel,
    out_type=[
        jax.ShapeDtypeStruct((N, D), jnp.float32),
        jax.ShapeDtypeStruct((NW * N,), jnp.float32),
    ],
    mesh=_mesh,
    scratch_types=[
        pltpu.VMEM((EPT,), jnp.int32),
        pltpu.VMEM((N,), jnp.float32),
        pltpu.VMEM((CK,), jnp.int32),
        pltpu.VMEM((CK, D), jnp.float32),
        pltpu.SemaphoreType.DMA,
    ],
    compiler_params=pltpu.CompilerParams(needs_layout_passes=False),
)
def _sc_emb_deg(x1, dst1, emb, h_out, degp_out, dst_v, acc, x_v, rows_v, sem):
    c = lax.axis_index("c")
    s = lax.axis_index("s")
    wid = c * NS + s

    pltpu.sync_copy(dst1.at[pl.ds(wid * EPT, EPT)], dst_v)

    def zero_body(j, carry):
        acc[pl.ds(VL * j, VL)] = jnp.zeros((VL,), jnp.float32)
        return carry

    lax.fori_loop(0, N // VL, zero_body, 0)

    ones = jnp.ones((VL,), jnp.float32)

    def deg_body(j, carry):
        iv = dst_v[pl.ds(VL * j, VL)]
        plsc.addupdate_scatter(acc, [iv], ones)
        return carry

    lax.fori_loop(0, EPT // VL, deg_body, 0)

    def wr_body(b, carry):
        pltpu.sync_copy(acc.at[pl.ds(b * _R, _R)],
                        degp_out.at[pl.ds((b * NW + wid) * _R, _R)])
        return carry

    lax.fori_loop(0, N // _R, wr_body, 0)

    nch = jnp.where(wid < NW - 1, 4, 1)

    def emb_body(j, carry):
        pltpu.sync_copy(x1.at[pl.ds(wid * 320 + j * CK, CK)], x_v)
        pltpu.async_copy(emb.at[x_v], rows_v, sem).wait()
        pltpu.sync_copy(rows_v, h_out.at[pl.ds(wid * 320 + j * CK, CK)])
        return carry

    lax.fori_loop(0, nch, emb_body, 0)


CK2 = 80
CPT2 = EPT // CK2
PH = 64
NB = 3


@functools.partial(
    pl.kernel,
    out_type=jax.ShapeDtypeStruct((NC, N, D), jnp.float32),
    mesh=_mesh,
    scratch_types=[
        pltpu.VMEM_SHARED((N, D), jnp.float32),
        pltpu.VMEM((PH * CK2,), jnp.int32),
        pltpu.VMEM((PH, CK2), jnp.int32),
        pltpu.VMEM((CK2, D), jnp.float32),
        pltpu.VMEM((CK2, D), jnp.float32),
        pltpu.VMEM((CK2, D), jnp.float32),
        pltpu.SemaphoreType.DMA,
        pltpu.SemaphoreType.DMA,
        pltpu.SemaphoreType.DMA,
    ],
)
def _sc_segsum(hs, src1, dst3p, zf80, out, acc, srcs, dsts, rb0, rb1, rb2,
               sem0, sem1, sem2):
    c = lax.axis_index("c")
    s = lax.axis_index("s")
    wid = c * NS + s
    bufs = (rb0, rb1, rb2)
    sems = (sem0, sem1, sem2)

    def fire(j, b):
        pltpu.async_copy(hs.at[srcs.at[pl.ds(j * CK2, CK2)]], bufs[b],
                         sems[b])

    def wait_scat(j, b):
        pltpu.make_async_copy(hs.at[srcs.at[pl.ds(j * CK2, CK2)]],
                              bufs[b], sems[b]).wait()
        pltpu.sync_copy(bufs[b], acc.at[dsts.at[j]], add=True)

    def run_phase(nchunks):
        for b in range(NB):
            fire(b, b)

        def body(g, carry):
            for b in range(NB):
                j = g * NB + b
                wait_scat(j, b)
                fire(j + NB, b)
            return carry

        lax.fori_loop(0, (nchunks - 4) // NB, body, 0)
        j0 = nchunks - 4
        wait_scat(j0, j0 % NB)
        fire(nchunks - 1, (nchunks - 1) % NB)
        for j in range(j0 + 1, nchunks):
            wait_scat(j, j % NB)

    pltpu.sync_copy(src1.at[pl.ds(wid * EPT, PH * CK2)], srcs)
    pltpu.sync_copy(dst3p.at[wid, pl.ds(0, PH)], dsts)

    pltpu.sync_copy(zf80, rb0)
    for k in range(7):
        pltpu.sync_copy(rb0, acc.at[pl.ds(s * ZR + k * CK2, CK2)])
    pltpu.sync_copy(rb0.at[pl.ds(0, 64)], acc.at[pl.ds(s * ZR + 560, 64)])

    @pl.when(s == NS - 1)
    def _():
        pltpu.sync_copy(rb0.at[pl.ds(0, N - NS * ZR)],
                        acc.at[pl.ds(NS * ZR, N - NS * ZR)])

    plsc.subcore_barrier()

    run_phase(PH)

    pltpu.sync_copy(src1.at[pl.ds(wid * EPT + PH * CK2, EPT - PH * CK2)],
                    srcs.at[pl.ds(0, EPT - PH * CK2)])
    pltpu.sync_copy(dst3p.at[wid, pl.ds(PH, PH)], dsts)

    run_phase(CPT2 - PH)

    plsc.subcore_barrier()
    pltpu.sync_copy(acc.at[pl.ds(s * ZR, ZR)], out.at[c, pl.ds(s * ZR, ZR)])

    @pl.when(s == NS - 1)
    def _():
        pltpu.sync_copy(acc.at[pl.ds(NS * ZR, N - NS * ZR)],
                        out.at[c, pl.ds(NS * ZR, N - NS * ZR)])


def _tc_scale_body(dps, h, hs1, dinv_out):
    dp = dps[...].reshape(NW, _R)
    ones = jnp.ones((NW, 1), jnp.float32)
    deg = lax.dot_general(dp, ones, (((0,), (0,)), ((), ())),
                          preferred_element_type=jnp.float32) + 1.0
    dcol = lax.rsqrt(deg)
    hs1[...] = h[...] * dcol
    dinv_out[...] = dcol


def _tc_mid_body(dinv, t0, t1, hs1, w1, b1, w2, hs2):
    dcol = dinv[...]
    aggh = (t0[...] + t1[...] + hs1[...]) * dcol
    o1 = jnp.dot(aggh, w1[...], preferred_element_type=jnp.float32) + b1[...]
    h2 = jnp.maximum(o1, 0.0)
    xw2 = jnp.dot(h2, w2[...], preferred_element_type=jnp.float32)
    hs2[...] = xw2 * dcol


def _tc_out_body(dinv, t0, t1, hs2, b2, out):
    out[...] = (t0[...] + t1[...] + hs2[...]) * dinv[...] + b2[...]


def _row_spec(w):
    return pl.BlockSpec((_R, w), lambda i: (i, 0))


def _full_spec(r, w):
    return pl.BlockSpec((r, w), lambda i: (0, 0))


def _tc_scale(dps, h):
    return pl.pallas_call(
        _tc_scale_body,
        grid=(N // _R,),
        in_specs=[pl.BlockSpec((1, NW, _R), lambda i: (i, 0, 0)),
                  _row_spec(D)],
        out_specs=[_row_spec(D), _row_spec(1)],
        out_shape=[jax.ShapeDtypeStruct((N, D), jnp.float32),
                   jax.ShapeDtypeStruct((N, 1), jnp.float32)],
    )(dps, h)


def _tc_mid(dinv, t0, t1, hs1, w1, b1, w2):
    return pl.pallas_call(
        _tc_mid_body,
        grid=(N // _R,),
        in_specs=[_row_spec(1), _row_spec(D), _row_spec(D), _row_spec(D),
                  _full_spec(D, H2), _full_spec(1, H2), _full_spec(H2, D)],
        out_specs=_row_spec(D),
        out_shape=jax.ShapeDtypeStruct((N, D), jnp.float32),
    )(dinv, t0, t1, hs1, w1, b1, w2)


def _tc_out(dinv, t0, t1, hs2, b2):
    return pl.pallas_call(
        _tc_out_body,
        grid=(N // _R,),
        in_specs=[_row_spec(1), _row_spec(D), _row_spec(D), _row_spec(D),
                  _full_spec(1, D)],
        out_specs=_row_spec(D),
        out_shape=jax.ShapeDtypeStruct((N, D), jnp.float32),
    )(dinv, t0, t1, hs2, b2)


def kernel(x, edge_index, emb_table, W1, b1, W2, b2):
    src1 = edge_index[0]
    dst1 = edge_index[1]
    dst3p = jnp.pad(edge_index[1].reshape(NW, CPT2, CK2),
                    ((0, 0), (0, 2 * PH - CPT2), (0, 0)))
    x1 = x[:, 0]
    zf80 = jnp.zeros((CK2, D), jnp.float32)

    h, degp = _sc_emb_deg(x1, dst1, emb_table)
    hs1, dinv = _tc_scale(degp.reshape(N // _R, NW, _R), h)

    t1 = _sc_segsum(hs1, src1, dst3p, zf80)
    hs2 = _tc_mid(dinv, t1[0], t1[1], hs1, W1, b1.reshape(1, H2), W2)
    t2 = _sc_segsum(hs2, src1, dst3p, zf80)
    return _tc_out(dinv, t2[0], t2[1], hs2, b2.reshape(1, D))
